# baseline (speedup 1.0000x reference)
import functools

import jax
import jax.numpy as jnp
from jax import lax
from jax.experimental import pallas as pl
from jax.experimental.pallas import tpu as pltpu


def _final_edge_kernel(t1_ref, ga2_ref, gb2_ref, bc2_ref, wg2_ref, ga3_ref,
                       gb3_ref, scal_ref, out_ref):
    c2 = jnp.maximum(ga2_ref[...] + gb2_ref[...] + bc2_ref[...], 0.0)
    g1 = jax.nn.sigmoid(t1_ref[...] + c2)
    r = jnp.sum(g1 * wg2_ref[...], axis=1, keepdims=True)
    c3 = jnp.maximum(ga3_ref[...] + gb3_ref[...] + scal_ref[0, 0], 0.0)
    out_ref[...] = jax.nn.sigmoid(r + scal_ref[0, 1] + c3)


def _final_edge(t1, ga2, gb2, bc2, wg2, ga3, gb3, bc3, bg2):
    E = t1.shape[0]
    BE = 3200
    scal = jnp.stack([bc3[0], bg2[0]]).reshape(1, 2)
    return pl.pallas_call(
        _final_edge_kernel,
        grid=(E // BE,),
        in_specs=[
            pl.BlockSpec((BE, 32), lambda i: (i, 0)),
            pl.BlockSpec((BE, 32), lambda i: (i, 0)),
            pl.BlockSpec((BE, 32), lambda i: (i, 0)),
            pl.BlockSpec((1, 32), lambda i: (0, 0)),
            pl.BlockSpec((1, 32), lambda i: (0, 0)),
            pl.BlockSpec((BE, 1), lambda i: (i, 0)),
            pl.BlockSpec((BE, 1), lambda i: (i, 0)),
            pl.BlockSpec((1, 2), lambda i: (0, 0)),
        ],
        out_specs=pl.BlockSpec((BE, 1), lambda i: (i, 0)),
        out_shape=jax.ShapeDtypeStruct((E, 1), jnp.float32),
    )(t1, ga2, gb2, jnp.reshape(bc2, (1, 32)), jnp.reshape(wg2, (1, 32)),
      ga3, gb3, scal)


def kernel(pre_node_embedding, edge_index, W1_self, W1_nbr, b1, Wc1, bc1,
           W2_self, W2_nbr, b2, Wc2, bc2, W3_self, W3_nbr, b3, Wc3, bc3,
           Wg1, bg1, Wg2, bg2):
    src = edge_index[0]
    dst = edge_index[1]
    x = pre_node_embedding
    n = x.shape[0]

    def segsum(v):
        return jnp.zeros((n, v.shape[1]), v.dtype).at[dst].add(v[src])

    h1 = jax.nn.relu(x @ W1_self + segsum(x) @ W1_nbr + b1)
    A1 = h1 @ Wc1[:128]
    B1 = h1 @ Wc1[128:]
    y2 = h1 @ W2_nbr
    self2 = h1 @ W2_self
    h2 = jax.nn.relu(self2 + segsum(y2) + b2)
    A2 = h2 @ Wc2[:64]
    B2 = h2 @ Wc2[64:]
    y3 = h2 @ W3_nbr
    self3 = h2 @ W3_self
    c1 = jax.nn.relu(A1[src] + B1[dst] + bc1)
    t1 = c1 @ Wg1 + bg1
    h3 = jax.nn.relu(self3 + segsum(y3) + b3)
    A3 = h3 @ Wc3[:32]
    B3 = h3 @ Wc3[32:]
    return _final_edge(t1, A2[src], B2[dst], bc2, Wg2, A3[src], B3[dst],
                       bc3, bg2)


# speedup vs baseline: 4.6964x; 4.6964x over previous
import functools

import jax
import jax.numpy as jnp
from jax import lax
from jax.experimental import pallas as pl
from jax.experimental.pallas import tpu as pltpu
from jax.experimental.pallas import tpu_sc as plsc

NC = 2
NS = 16
NW = NC * NS
CH = 80


def _sc_mesh():
    return plsc.VectorSubcoreMesh(core_axis_name="c", subcore_axis_name="s")


def _stage1(npad, e):
    EW = e // NW
    K = EW // CH
    RT = npad // NS

    def body(tbl_h, src_h, dst_h, zeros_h, outp_h, sidx, didx, rows, acc,
             sem):
        cid = lax.axis_index("c")
        sid = lax.axis_index("s")
        base = (cid * NS + sid) * EW
        sl = pl.ds(sid * RT, RT)
        pltpu.sync_copy(zeros_h.at[sl], acc.at[sl])
        plsc.subcore_barrier()

        def step(k, c):
            e0 = base + k * CH
            pltpu.sync_copy(src_h.at[pl.ds(e0, CH)], sidx)
            pltpu.sync_copy(dst_h.at[pl.ds(e0, CH)], didx)
            pltpu.async_copy(tbl_h.at[sidx], rows, sem).wait()
            pltpu.sync_copy(rows, acc.at[didx], add=True)
            return c
        lax.fori_loop(0, K, step, 0)
        plsc.subcore_barrier()
        pltpu.sync_copy(acc.at[pl.ds(sid * RT, RT)],
                        outp_h.at[pl.ds(cid * npad + sid * RT, RT)])

    return pl.kernel(
        body,
        out_type=(jax.ShapeDtypeStruct((NC * npad, 128), jnp.float32),),
        mesh=_sc_mesh(),
        scratch_types=(pltpu.VMEM((CH,), jnp.int32),
                       pltpu.VMEM((CH,), jnp.int32),
                       pltpu.VMEM((CH, 128), jnp.float32),
                       pltpu.VMEM_SHARED((npad, 128), jnp.float32),
                       pltpu.SemaphoreType.DMA))


def _stage2(npad, e):
    EW = e // NW
    K = EW // CH
    RT = npad // NS

    def body(h_h, t_h, src_h, dst_h, zeros_h, outp_h, outs_h, outd_h,
             sidx, didx, rows, rowS, rowD, acc, sem):
        cid = lax.axis_index("c")
        sid = lax.axis_index("s")
        base = (cid * NS + sid) * EW
        sl = pl.ds(sid * RT, RT)
        pltpu.sync_copy(zeros_h.at[sl], acc.at[sl])
        plsc.subcore_barrier()

        def step(k, c):
            e0 = base + k * CH
            pltpu.sync_copy(src_h.at[pl.ds(e0, CH)], sidx)
            pltpu.sync_copy(dst_h.at[pl.ds(e0, CH)], didx)
            pltpu.async_copy(h_h.at[sidx], rows, sem).wait()
            pltpu.sync_copy(rows, acc.at[didx], add=True)
            pltpu.async_copy(t_h.at[sidx], rowS, sem).wait()
            pltpu.sync_copy(rowS, outs_h.at[pl.ds(e0, CH)])
            pltpu.async_copy(t_h.at[didx], rowD, sem).wait()
            pltpu.sync_copy(rowD, outd_h.at[pl.ds(e0, CH)])
            return c
        lax.fori_loop(0, K, step, 0)
        plsc.subcore_barrier()
        pltpu.sync_copy(acc.at[pl.ds(sid * RT, RT)],
                        outp_h.at[pl.ds(cid * npad + sid * RT, RT)])

    return pl.kernel(
        body,
        out_type=(jax.ShapeDtypeStruct((NC * npad, 128), jnp.float32),
                  jax.ShapeDtypeStruct((e, 128), jnp.float32),
                  jax.ShapeDtypeStruct((e, 128), jnp.float32)),
        mesh=_sc_mesh(),
        scratch_types=(pltpu.VMEM((CH,), jnp.int32),
                       pltpu.VMEM((CH,), jnp.int32),
                       pltpu.VMEM((CH, 128), jnp.float32),
                       pltpu.VMEM((CH, 128), jnp.float32),
                       pltpu.VMEM((CH, 128), jnp.float32),
                       pltpu.VMEM_SHARED((npad, 128), jnp.float32),
                       pltpu.SemaphoreType.DMA))


def _stage3(npad, e):
    EW = e // NW
    K = EW // CH
    RT = npad // NS

    def body(u_h, src_h, dst_h, zeros_h, outp_h, outs_h, outd_h,
             sidx, didx, rows, rowD, acc, sem):
        cid = lax.axis_index("c")
        sid = lax.axis_index("s")
        base = (cid * NS + sid) * EW
        sl = pl.ds(sid * RT, RT)
        pltpu.sync_copy(zeros_h.at[sl], acc.at[sl])
        plsc.subcore_barrier()

        def step(k, c):
            e0 = base + k * CH
            pltpu.sync_copy(src_h.at[pl.ds(e0, CH)], sidx)
            pltpu.sync_copy(dst_h.at[pl.ds(e0, CH)], didx)
            pltpu.async_copy(u_h.at[sidx], rows, sem).wait()
            pltpu.sync_copy(rows, acc.at[didx], add=True)
            pltpu.sync_copy(rows, outs_h.at[pl.ds(e0, CH)])
            pltpu.async_copy(u_h.at[didx], rowD, sem).wait()
            pltpu.sync_copy(rowD, outd_h.at[pl.ds(e0, CH)])
            return c
        lax.fori_loop(0, K, step, 0)
        plsc.subcore_barrier()
        pltpu.sync_copy(acc.at[pl.ds(sid * RT, RT)],
                        outp_h.at[pl.ds(cid * npad + sid * RT, RT)])

    return pl.kernel(
        body,
        out_type=(jax.ShapeDtypeStruct((NC * npad, 128), jnp.float32),
                  jax.ShapeDtypeStruct((e, 128), jnp.float32),
                  jax.ShapeDtypeStruct((e, 128), jnp.float32)),
        mesh=_sc_mesh(),
        scratch_types=(pltpu.VMEM((CH,), jnp.int32),
                       pltpu.VMEM((CH,), jnp.int32),
                       pltpu.VMEM((CH, 128), jnp.float32),
                       pltpu.VMEM((CH, 128), jnp.float32),
                       pltpu.VMEM_SHARED((npad, 128), jnp.float32),
                       pltpu.SemaphoreType.DMA))


def _stage4(npad, e):
    EW = e // NW
    K = EW // CH

    def body(a3_h, b3_h, src_h, dst_h, out_h, ta, tb, sidx, didx, ob, sem):
        cid = lax.axis_index("c")
        sid = lax.axis_index("s")
        base = (cid * NS + sid) * EW
        pltpu.sync_copy(a3_h, ta)
        pltpu.sync_copy(b3_h, tb)

        def step(k, c):
            e0 = base + k * CH
            pltpu.sync_copy(src_h.at[pl.ds(e0, CH)], sidx)
            pltpu.sync_copy(dst_h.at[pl.ds(e0, CH)], didx)
            for j in range(CH // 16):
                ii = sidx[pl.ds(j * 16, 16)]
                jj = didx[pl.ds(j * 16, 16)]
                va = plsc.load_gather(ta, [ii])
                vb = plsc.load_gather(tb, [jj])
                ob[pl.ds(j * 16, 16)] = va + vb
            pltpu.sync_copy(ob, out_h.at[pl.ds(e0, CH)])
            return c
        lax.fori_loop(0, K, step, 0)

    return pl.kernel(
        body,
        out_type=(jax.ShapeDtypeStruct((e,), jnp.float32),),
        mesh=_sc_mesh(),
        compiler_params=pltpu.CompilerParams(needs_layout_passes=False),
        scratch_types=(pltpu.VMEM((npad,), jnp.float32),
                       pltpu.VMEM((npad,), jnp.float32),
                       pltpu.VMEM((CH,), jnp.int32),
                       pltpu.VMEM((CH,), jnp.int32),
                       pltpu.VMEM((CH,), jnp.float32),
                       pltpu.SemaphoreType.DMA))


def _edge_kernel(gs2_ref, gd2_ref, gs3_ref, gd3_ref, g3_ref, bc1_ref,
                 wg1_ref, bg1_ref, bc2_ref, wg2_ref, scal_ref, out_ref):
    c1 = jnp.maximum(gs2_ref[:, :64] + gd2_ref[:, 64:] + bc1_ref[...], 0.0)
    t1 = jnp.dot(c1, wg1_ref[...],
                 preferred_element_type=jnp.float32) + bg1_ref[...]
    c2 = jnp.maximum(gs3_ref[:, 64:96] + gd3_ref[:, 96:128] + bc2_ref[...],
                     0.0)
    g1 = jax.nn.sigmoid(t1 + c2)
    r = jnp.sum(g1 * wg2_ref[...], axis=1, keepdims=True)
    c3 = jnp.maximum(g3_ref[...] + scal_ref[0, 0], 0.0)
    out_ref[...] = jax.nn.sigmoid(r + scal_ref[0, 1] + c3)


def _edge_stage(gs2, gd2, gs3, gd3, g3, bc1, wg1, bg1, bc2, wg2, bc3, bg2):
    e = gs2.shape[0]
    BE = 3200
    scal = jnp.stack([bc3[0], bg2[0]]).reshape(1, 2)
    return pl.pallas_call(
        _edge_kernel,
        grid=(e // BE,),
        in_specs=[
            pl.BlockSpec((BE, 128), lambda i: (i, 0)),
            pl.BlockSpec((BE, 128), lambda i: (i, 0)),
            pl.BlockSpec((BE, 128), lambda i: (i, 0)),
            pl.BlockSpec((BE, 128), lambda i: (i, 0)),
            pl.BlockSpec((BE, 1), lambda i: (i, 0)),
            pl.BlockSpec((1, 64), lambda i: (0, 0)),
            pl.BlockSpec((64, 32), lambda i: (0, 0)),
            pl.BlockSpec((1, 32), lambda i: (0, 0)),
            pl.BlockSpec((1, 32), lambda i: (0, 0)),
            pl.BlockSpec((1, 32), lambda i: (0, 0)),
            pl.BlockSpec((1, 2), lambda i: (0, 0)),
        ],
        out_specs=pl.BlockSpec((BE, 1), lambda i: (i, 0)),
        out_shape=jax.ShapeDtypeStruct((e, 1), jnp.float32),
    )(gs2, gd2, gs3, gd3, g3, jnp.reshape(bc1, (1, 64)), wg1,
      jnp.reshape(bg1, (1, 32)), jnp.reshape(bc2, (1, 32)),
      jnp.reshape(wg2, (1, 32)), scal)


def kernel(pre_node_embedding, edge_index, W1_self, W1_nbr, b1, Wc1, bc1,
           W2_self, W2_nbr, b2, Wc2, bc2, W3_self, W3_nbr, b3, Wc3, bc3,
           Wg1, bg1, Wg2, bg2):
    x = pre_node_embedding
    n = x.shape[0]
    e = edge_index.shape[1]
    src = jnp.asarray(edge_index[0])
    dst = jnp.asarray(edge_index[1])
    npad = ((n + NS * 8 - 1) // (NS * 8)) * (NS * 8)
    xp = jnp.pad(x, ((0, npad - n), (0, 0)))
    z128 = jnp.zeros((npad, 128), jnp.float32)

    p1 = _stage1(npad, e)(xp, src, dst, z128)[0]
    agg1 = p1[:npad] + p1[npad:]

    h1 = jax.nn.relu(xp @ W1_self + agg1 @ W1_nbr + b1)
    T2 = jnp.concatenate([h1 @ Wc1[:128], h1 @ Wc1[128:]], axis=1)
    self2 = h1 @ W2_self

    p2, gS2, gD2 = _stage2(npad, e)(h1, T2, src, dst, z128)
    agg2 = (p2[:npad] + p2[npad:]) @ W2_nbr

    h2 = jax.nn.relu(self2 + agg2 + b2)
    U3 = jnp.concatenate([h2, h2 @ Wc2[:64], h2 @ Wc2[64:]], axis=1)
    self3 = h2 @ W3_self

    p3, gS3, gD3 = _stage3(npad, e)(U3, src, dst, z128)
    agg3 = (p3[:npad, :64] + p3[npad:, :64]) @ W3_nbr

    h3 = jax.nn.relu(self3 + agg3 + b3)
    A3 = h3 @ Wc3[:32]
    B3 = h3 @ Wc3[32:]

    g3 = _stage4(npad, e)(jnp.reshape(A3, (npad,)), jnp.reshape(B3, (npad,)),
                          src, dst)[0]

    return _edge_stage(gS2, gD2, gS3, gD3, jnp.reshape(g3, (e, 1)),
                       bc1, Wg1, bg1, bc2, Wg2, bc3, bg2)


# speedup vs baseline: 5.2533x; 1.1186x over previous
import functools

import jax
import jax.numpy as jnp
from jax import lax
from jax.experimental import pallas as pl
from jax.experimental.pallas import tpu as pltpu
from jax.experimental.pallas import tpu_sc as plsc

NC = 2
NS = 16
NW = NC * NS
CH = 80


def _sc_mesh():
    return plsc.VectorSubcoreMesh(core_axis_name="c", subcore_axis_name="s")


def _stage1(npad, e):
    EW = e // NW
    K = EW // CH
    RT = npad // NS

    def body(tbl_h, src_h, dst_h, zeros_h, outp_h, sidx, didx, rows, acc,
             sem):
        cid = lax.axis_index("c")
        sid = lax.axis_index("s")
        base = (cid * NS + sid) * EW
        sl = pl.ds(sid * RT, RT)
        pltpu.sync_copy(zeros_h.at[sl], acc.at[sl])
        plsc.subcore_barrier()

        def step(k, c):
            e0 = base + k * CH
            pltpu.sync_copy(src_h.at[pl.ds(e0, CH)], sidx)
            pltpu.sync_copy(dst_h.at[pl.ds(e0, CH)], didx)
            pltpu.async_copy(tbl_h.at[sidx], rows, sem).wait()
            pltpu.sync_copy(rows, acc.at[didx], add=True)
            return c
        lax.fori_loop(0, K, step, 0)
        plsc.subcore_barrier()
        pltpu.sync_copy(acc.at[pl.ds(sid * RT, RT)],
                        outp_h.at[pl.ds(cid * npad + sid * RT, RT)])

    return pl.kernel(
        body,
        out_type=(jax.ShapeDtypeStruct((NC * npad, 128), jnp.float32),),
        mesh=_sc_mesh(),
        scratch_types=(pltpu.VMEM((CH,), jnp.int32),
                       pltpu.VMEM((CH,), jnp.int32),
                       pltpu.VMEM((CH, 128), jnp.float32),
                       pltpu.VMEM_SHARED((npad, 128), jnp.float32),
                       pltpu.SemaphoreType.DMA))


def _stage2(npad, e):
    EW = e // NW
    K = EW // CH
    RT = npad // NS

    def body(h_h, t_h, src_h, dst_h, zeros_h, outp_h, outs_h, outd_h,
             sidx, didx, rows, rowS, rowD, acc, sem):
        cid = lax.axis_index("c")
        sid = lax.axis_index("s")
        base = (cid * NS + sid) * EW
        sl = pl.ds(sid * RT, RT)
        pltpu.sync_copy(zeros_h.at[sl], acc.at[sl])
        plsc.subcore_barrier()

        def step(k, c):
            e0 = base + k * CH
            pltpu.sync_copy(src_h.at[pl.ds(e0, CH)], sidx)
            pltpu.sync_copy(dst_h.at[pl.ds(e0, CH)], didx)
            pltpu.async_copy(h_h.at[sidx], rows, sem).wait()
            pltpu.sync_copy(rows, acc.at[didx], add=True)
            pltpu.async_copy(t_h.at[sidx], rowS, sem).wait()
            pltpu.sync_copy(rowS, outs_h.at[pl.ds(e0, CH)])
            pltpu.async_copy(t_h.at[didx], rowD, sem).wait()
            pltpu.sync_copy(rowD, outd_h.at[pl.ds(e0, CH)])
            return c
        lax.fori_loop(0, K, step, 0)
        plsc.subcore_barrier()
        pltpu.sync_copy(acc.at[pl.ds(sid * RT, RT)],
                        outp_h.at[pl.ds(cid * npad + sid * RT, RT)])

    return pl.kernel(
        body,
        out_type=(jax.ShapeDtypeStruct((NC * npad, 128), jnp.float32),
                  jax.ShapeDtypeStruct((e, 128), jnp.float32),
                  jax.ShapeDtypeStruct((e, 128), jnp.float32)),
        mesh=_sc_mesh(),
        scratch_types=(pltpu.VMEM((CH,), jnp.int32),
                       pltpu.VMEM((CH,), jnp.int32),
                       pltpu.VMEM((CH, 128), jnp.float32),
                       pltpu.VMEM((CH, 128), jnp.float32),
                       pltpu.VMEM((CH, 128), jnp.float32),
                       pltpu.VMEM_SHARED((npad, 128), jnp.float32),
                       pltpu.SemaphoreType.DMA))


def _stage3(npad, e):
    EW = e // NW
    K = EW // CH
    RT = npad // NS

    def body(u_h, src_h, dst_h, zeros_h, outp_h, outs_h, outd_h,
             sidx, didx, rows, rowD, acc, sem):
        cid = lax.axis_index("c")
        sid = lax.axis_index("s")
        base = (cid * NS + sid) * EW
        sl = pl.ds(sid * RT, RT)
        pltpu.sync_copy(zeros_h.at[sl], acc.at[sl])
        plsc.subcore_barrier()

        def step(k, c):
            e0 = base + k * CH
            pltpu.sync_copy(src_h.at[pl.ds(e0, CH)], sidx)
            pltpu.sync_copy(dst_h.at[pl.ds(e0, CH)], didx)
            pltpu.async_copy(u_h.at[sidx], rows, sem).wait()
            pltpu.sync_copy(rows, acc.at[didx], add=True)
            pltpu.sync_copy(rows, outs_h.at[pl.ds(e0, CH)])
            pltpu.async_copy(u_h.at[didx], rowD, sem).wait()
            pltpu.sync_copy(rowD, outd_h.at[pl.ds(e0, CH)])
            return c
        lax.fori_loop(0, K, step, 0)
        plsc.subcore_barrier()
        pltpu.sync_copy(acc.at[pl.ds(sid * RT, RT)],
                        outp_h.at[pl.ds(cid * npad + sid * RT, RT)])

    return pl.kernel(
        body,
        out_type=(jax.ShapeDtypeStruct((NC * npad, 128), jnp.float32),
                  jax.ShapeDtypeStruct((e, 128), jnp.float32),
                  jax.ShapeDtypeStruct((e, 128), jnp.float32)),
        mesh=_sc_mesh(),
        scratch_types=(pltpu.VMEM((CH,), jnp.int32),
                       pltpu.VMEM((CH,), jnp.int32),
                       pltpu.VMEM((CH, 128), jnp.float32),
                       pltpu.VMEM((CH, 128), jnp.float32),
                       pltpu.VMEM_SHARED((npad, 128), jnp.float32),
                       pltpu.SemaphoreType.DMA))


def _seg_pipelined(npad, e, mode):
    EW = e // NW
    K = EW // CH
    RT = npad // NS

    def body(*refs):
        i = 0
        if mode == 1:
            tbl = refs[i]; i += 1
        elif mode == 2:
            h_h = refs[i]; t_h = refs[i + 1]; i += 2
        else:
            u_h = refs[i]; i += 1
        src3 = refs[i]; dst3 = refs[i + 1]; zeros_h = refs[i + 2]; i += 3
        outp_h = refs[i]; i += 1
        if mode != 1:
            outs_h = refs[i]; outd_h = refs[i + 1]; i += 2
        sidx2 = refs[i]; didx2 = refs[i + 1]; i += 2
        rows = refs[i]; i += 1
        if mode == 2:
            rowS = refs[i]; i += 1
        if mode != 1:
            rowD = refs[i]; i += 1
        acc = refs[i]; semg = refs[i + 1]

        cid = lax.axis_index("c")
        sid = lax.axis_index("s")
        wid = cid * NS + sid
        base = wid * EW
        sl = pl.ds(sid * RT, RT)
        pltpu.sync_copy(zeros_h.at[sl], acc.at[sl])
        plsc.subcore_barrier()

        def step(k, c):
            e0 = base + k * CH
            pltpu.sync_copy(src3.at[wid, k], sidx2)
            pltpu.sync_copy(dst3.at[wid, k], didx2)
            si = sidx2
            di = didx2
            if mode == 1:
                pltpu.async_copy(tbl.at[si], rows, semg)
                pltpu.make_async_copy(tbl.at[si], rows, semg).wait()
                pltpu.sync_copy(rows, acc.at[di], add=True)
            elif mode == 2:
                pltpu.async_copy(h_h.at[si], rows, semg)
                pltpu.async_copy(t_h.at[si], rowS, semg)
                pltpu.async_copy(t_h.at[di], rowD, semg)
                pltpu.make_async_copy(h_h.at[si], rows, semg).wait()
                pltpu.make_async_copy(t_h.at[si], rowS, semg).wait()
                pltpu.make_async_copy(t_h.at[di], rowD, semg).wait()
                pltpu.sync_copy(rows, acc.at[di], add=True)
                pltpu.sync_copy(rowS, outs_h.at[pl.ds(e0, CH)])
                pltpu.sync_copy(rowD, outd_h.at[pl.ds(e0, CH)])
            else:
                pltpu.async_copy(u_h.at[si], rows, semg)
                pltpu.async_copy(u_h.at[di], rowD, semg)
                pltpu.make_async_copy(u_h.at[si], rows, semg).wait()
                pltpu.make_async_copy(u_h.at[di], rowD, semg).wait()
                pltpu.sync_copy(rows, acc.at[di], add=True)
                pltpu.sync_copy(rows, outs_h.at[pl.ds(e0, CH)])
                pltpu.sync_copy(rowD, outd_h.at[pl.ds(e0, CH)])
            return c
        lax.fori_loop(0, K, step, 0)
        plsc.subcore_barrier()
        pltpu.sync_copy(acc.at[pl.ds(sid * RT, RT)],
                        outp_h.at[pl.ds(cid * npad + sid * RT, RT)])

    out_type = [jax.ShapeDtypeStruct((NC * npad, 128), jnp.float32)]
    if mode != 1:
        out_type.append(jax.ShapeDtypeStruct((e, 128), jnp.float32))
        out_type.append(jax.ShapeDtypeStruct((e, 128), jnp.float32))
    scratch = [pltpu.VMEM((CH,), jnp.int32), pltpu.VMEM((CH,), jnp.int32),
               pltpu.VMEM((CH, 128), jnp.float32)]
    if mode == 2:
        scratch.append(pltpu.VMEM((CH, 128), jnp.float32))
    if mode != 1:
        scratch.append(pltpu.VMEM((CH, 128), jnp.float32))
    scratch += [pltpu.VMEM_SHARED((npad, 128), jnp.float32),
                pltpu.SemaphoreType.DMA]
    return pl.kernel(body, out_type=tuple(out_type), mesh=_sc_mesh(),
                     scratch_types=tuple(scratch))


def _stage4(npad, e):
    EW = e // NW
    K = EW // CH

    def body(a3_h, b3_h, src_h, dst_h, out_h, ta, tb, sidx, didx, ob, sem):
        cid = lax.axis_index("c")
        sid = lax.axis_index("s")
        base = (cid * NS + sid) * EW
        pltpu.sync_copy(a3_h, ta)
        pltpu.sync_copy(b3_h, tb)

        def step(k, c):
            e0 = base + k * CH
            pltpu.sync_copy(src_h.at[pl.ds(e0, CH)], sidx)
            pltpu.sync_copy(dst_h.at[pl.ds(e0, CH)], didx)
            for j in range(CH // 16):
                ii = sidx[pl.ds(j * 16, 16)]
                jj = didx[pl.ds(j * 16, 16)]
                va = plsc.load_gather(ta, [ii])
                vb = plsc.load_gather(tb, [jj])
                ob[pl.ds(j * 16, 16)] = va + vb
            pltpu.sync_copy(ob, out_h.at[pl.ds(e0, CH)])
            return c
        lax.fori_loop(0, K, step, 0)

    return pl.kernel(
        body,
        out_type=(jax.ShapeDtypeStruct((e,), jnp.float32),),
        mesh=_sc_mesh(),
        compiler_params=pltpu.CompilerParams(needs_layout_passes=False),
        scratch_types=(pltpu.VMEM((npad,), jnp.float32),
                       pltpu.VMEM((npad,), jnp.float32),
                       pltpu.VMEM((CH,), jnp.int32),
                       pltpu.VMEM((CH,), jnp.int32),
                       pltpu.VMEM((CH,), jnp.float32),
                       pltpu.SemaphoreType.DMA))


def _edge_kernel(gs2_ref, gd2_ref, gs3_ref, gd3_ref, g3_ref, bc1_ref,
                 wg1_ref, bg1_ref, bc2_ref, wg2_ref, scal_ref, out_ref):
    c1 = jnp.maximum(gs2_ref[:, :64] + gd2_ref[:, 64:] + bc1_ref[...], 0.0)
    t1 = jnp.dot(c1, wg1_ref[...],
                 preferred_element_type=jnp.float32) + bg1_ref[...]
    c2 = jnp.maximum(gs3_ref[:, 64:96] + gd3_ref[:, 96:128] + bc2_ref[...],
                     0.0)
    g1 = jax.nn.sigmoid(t1 + c2)
    r = jnp.sum(g1 * wg2_ref[...], axis=1, keepdims=True)
    c3 = jnp.maximum(g3_ref[...] + scal_ref[0, 0], 0.0)
    out_ref[...] = jax.nn.sigmoid(r + scal_ref[0, 1] + c3)


def _edge_stage(gs2, gd2, gs3, gd3, g3, bc1, wg1, bg1, bc2, wg2, bc3, bg2):
    e = gs2.shape[0]
    BE = 3200
    scal = jnp.stack([bc3[0], bg2[0]]).reshape(1, 2)
    return pl.pallas_call(
        _edge_kernel,
        grid=(e // BE,),
        in_specs=[
            pl.BlockSpec((BE, 128), lambda i: (i, 0)),
            pl.BlockSpec((BE, 128), lambda i: (i, 0)),
            pl.BlockSpec((BE, 128), lambda i: (i, 0)),
            pl.BlockSpec((BE, 128), lambda i: (i, 0)),
            pl.BlockSpec((BE, 1), lambda i: (i, 0)),
            pl.BlockSpec((1, 64), lambda i: (0, 0)),
            pl.BlockSpec((64, 32), lambda i: (0, 0)),
            pl.BlockSpec((1, 32), lambda i: (0, 0)),
            pl.BlockSpec((1, 32), lambda i: (0, 0)),
            pl.BlockSpec((1, 32), lambda i: (0, 0)),
            pl.BlockSpec((1, 2), lambda i: (0, 0)),
        ],
        out_specs=pl.BlockSpec((BE, 1), lambda i: (i, 0)),
        out_shape=jax.ShapeDtypeStruct((e, 1), jnp.float32),
    )(gs2, gd2, gs3, gd3, g3, jnp.reshape(bc1, (1, 64)), wg1,
      jnp.reshape(bg1, (1, 32)), jnp.reshape(bc2, (1, 32)),
      jnp.reshape(wg2, (1, 32)), scal)


def kernel(pre_node_embedding, edge_index, W1_self, W1_nbr, b1, Wc1, bc1,
           W2_self, W2_nbr, b2, Wc2, bc2, W3_self, W3_nbr, b3, Wc3, bc3,
           Wg1, bg1, Wg2, bg2):
    x = pre_node_embedding
    n = x.shape[0]
    e = edge_index.shape[1]
    src = jnp.asarray(edge_index[0])
    dst = jnp.asarray(edge_index[1])
    npad = ((n + NS * 8 - 1) // (NS * 8)) * (NS * 8)
    xp = jnp.pad(x, ((0, npad - n), (0, 0)))
    z128 = jnp.zeros((npad, 128), jnp.float32)

    K = e // NW // CH
    src3 = jnp.reshape(src, (NW, K, CH))
    dst3 = jnp.reshape(dst, (NW, K, CH))

    p1 = _seg_pipelined(npad, e, 1)(xp, src3, dst3, z128)[0]
    agg1 = p1[:npad] + p1[npad:]

    h1 = jax.nn.relu(xp @ W1_self + agg1 @ W1_nbr + b1)
    T2 = jnp.concatenate([h1 @ Wc1[:128], h1 @ Wc1[128:]], axis=1)
    self2 = h1 @ W2_self

    p2, gS2, gD2 = _seg_pipelined(npad, e, 2)(h1, T2, src3, dst3, z128)
    agg2 = (p2[:npad] + p2[npad:]) @ W2_nbr

    h2 = jax.nn.relu(self2 + agg2 + b2)
    U3 = jnp.concatenate([h2, h2 @ Wc2[:64], h2 @ Wc2[64:]], axis=1)
    self3 = h2 @ W3_self

    p3, gS3, gD3 = _seg_pipelined(npad, e, 3)(U3, src3, dst3, z128)
    agg3 = (p3[:npad, :64] + p3[npad:, :64]) @ W3_nbr

    h3 = jax.nn.relu(self3 + agg3 + b3)
    A3 = h3 @ Wc3[:32]
    B3 = h3 @ Wc3[32:]

    g3 = _stage4(npad, e)(jnp.reshape(A3, (npad,)), jnp.reshape(B3, (npad,)),
                          src, dst)[0]

    return _edge_stage(gS2, gD2, gS3, gD3, jnp.reshape(g3, (e, 1)),
                       bc1, Wg1, bg1, bc2, Wg2, bc3, bg2)
